# baseline (speedup 1.0000x reference)
import functools

import jax
import jax.numpy as jnp
from jax import lax
from jax.experimental import pallas as pl
from jax.experimental.pallas import tpu as pltpu
from jax.experimental.pallas import tpu_sc as plsc

_NODE_SIZE = 100000
_PATH_SIZE = 100
_D = 128
_B = 16384
_L = 16
_NC, _NS = 2, 16
_NW = _NC * _NS
_BPW = _B // _NW
_C = 128
_NCHUNK = _BPW // _C
_PROWS = 8
_PPAD = _PROWS * _NS


def _lane_perm(x, idx):
    dn = lax.GatherDimensionNumbers(offset_dims=(), collapsed_slice_dims=(0,),
                                    start_index_map=(0,))
    return lax.gather(x, idx[:, None], dn, slice_sizes=(1,),
                      mode=lax.GatherScatterMode.PROMISE_IN_BOUNDS)


def _sigmoid(x):
    return 1.0 / (1.0 + jnp.exp(-x))


def _hin2vec_body(start_hbm, end_hbm, path_hbm, node_hbm, ptab_hbm, w_hbm,
                  b_hbm, out_hbm, sidx, eidx, pidx, srows, erows, pwtab,
                  wv, bv, ptmp, ptfull, ostage, pw_shared, sem0, sem1):
    cid = lax.axis_index("c")
    sid = lax.axis_index("s")
    wid = sid * _NC + cid
    base = wid * _BPW

    pltpu.sync_copy(w_hbm, wv)
    pltpu.sync_copy(b_hbm, bv)
    pltpu.sync_copy(start_hbm.at[pl.ds(base, _BPW)], sidx)
    pltpu.sync_copy(end_hbm.at[pl.ds(base, _BPW)], eidx)
    pltpu.sync_copy(path_hbm.at[pl.ds(base, _BPW)], pidx.at[pl.ds(0, _BPW)])

    wregs = [wv[pl.ds(16 * j, _L)] for j in range(_D // _L)]
    bvec = bv[...]
    lane = lax.iota(jnp.int32, _L)
    perm = [(lane + sh) & (_L - 1) for sh in (8, 4, 2, 1)]

    prow0 = sid * _PROWS
    pltpu.sync_copy(ptab_hbm, ptfull)
    for r in range(_PROWS):
        rr = jnp.minimum(prow0 + r, _PATH_SIZE - 1)
        for j in range(_D // _L):
            v = ptfull[rr, pl.ds(16 * j, _L)]
            ptmp[r, pl.ds(16 * j, _L)] = _sigmoid(v) * wregs[j]
    pltpu.sync_copy(ptmp, pw_shared.at[pl.ds(prow0, _PROWS)])
    plsc.subcore_barrier()
    pltpu.sync_copy(pw_shared, pwtab)

    sems = [sem0, sem1]

    def fire(c):
        half = c % 2
        sl = pl.ds(c * _C, _C)
        sem = sems[half]
        hs = pl.ds(half * _C, _C)
        return (
            pltpu.async_copy(node_hbm.at[sidx.at[sl]], srows.at[hs], sem),
            pltpu.async_copy(node_hbm.at[eidx.at[sl]], erows.at[hs], sem),
        )

    pending = fire(0)
    for c in range(_NCHUNK):
        nxt = fire(c + 1) if c + 1 < _NCHUNK else None
        for cp in pending:
            cp.wait()
        pending = nxt
        rbase = (c % 2) * _C

        def group_body(g, _):
            def row_body(i, resvec):
                row = rbase + g * _L + i
                pvec = pidx[pl.ds(c * _C + g * _L + i, _L)]
                prow = pvec[0]
                acc = jnp.zeros((_L,), jnp.float32)
                for j in range(_D // _L):
                    s = srows[row, pl.ds(16 * j, _L)]
                    e = erows[row, pl.ds(16 * j, _L)]
                    pw = pwtab[prow, pl.ds(16 * j, _L)]
                    acc = acc + (s * e) * pw
                for pidx_v in perm:
                    acc = acc + _lane_perm(acc, pidx_v)
                return jnp.where(lane == i, acc, resvec)

            resvec = lax.fori_loop(0, _L, row_body,
                                   jnp.zeros((_L,), jnp.float32))
            outv = _sigmoid(resvec + bvec)
            ostage[pl.ds(c * _C + g * _L, _L)] = outv
            return 0

        lax.fori_loop(0, _C // _L, group_body, 0)

    pltpu.sync_copy(ostage, out_hbm.at[pl.ds(base, _BPW)])


@jax.jit
def _hin2vec_sc(start_i, end_i, path_i, node_table, ptab, w_flat, b_vec):
    mesh = plsc.VectorSubcoreMesh(core_axis_name="c", subcore_axis_name="s")
    f = functools.partial(
        pl.kernel,
        mesh=mesh,
        out_type=jax.ShapeDtypeStruct((_B,), jnp.float32),
        scratch_types=[
            pltpu.VMEM((_BPW,), jnp.int32),
            pltpu.VMEM((_BPW,), jnp.int32),
            pltpu.VMEM((_BPW + _L,), jnp.int32),
            pltpu.VMEM((2 * _C, _D), jnp.float32),
            pltpu.VMEM((2 * _C, _D), jnp.float32),
            pltpu.VMEM((_PPAD, _D), jnp.float32),
            pltpu.VMEM((_D,), jnp.float32),
            pltpu.VMEM((_L,), jnp.float32),
            pltpu.VMEM((_PROWS, _D), jnp.float32),
            pltpu.VMEM((_PATH_SIZE, _D), jnp.float32),
            pltpu.VMEM((_BPW,), jnp.float32),
            pltpu.VMEM_SHARED((_PPAD, _D), jnp.float32),
            pltpu.SemaphoreType.DMA,
            pltpu.SemaphoreType.DMA,
        ],
    )(_hin2vec_body)
    return f(start_i, end_i, path_i, node_table, ptab, w_flat, b_vec)


def _as_i32(x):
    return x if x.dtype == jnp.int32 else x.astype(jnp.int32)


def kernel(start_node, end_node, path, node_table, path_table, W, b):
    w_flat = W.reshape(_D)
    b_vec = jnp.broadcast_to(b.reshape(()), (_L,)).astype(jnp.float32)
    out = _hin2vec_sc(_as_i32(start_node), _as_i32(end_node), _as_i32(path),
                      node_table, path_table, w_flat, b_vec)
    return out.reshape(_B, 1)


# speedup vs baseline: 1.0223x; 1.0223x over previous
import functools

import jax
import jax.numpy as jnp
from jax import lax
from jax.experimental import pallas as pl
from jax.experimental.pallas import tpu as pltpu
from jax.experimental.pallas import tpu_sc as plsc

_NODE_SIZE = 100000
_PATH_SIZE = 100
_D = 128
_B = 16384
_L = 16
_NC, _NS = 2, 16
_NW = _NC * _NS
_BPW = _B // _NW
_C = 128
_NCHUNK = _BPW // _C
_PROWS = 8
_PPAD = _PROWS * _NS


def _lane_perm(x, idx):
    dn = lax.GatherDimensionNumbers(offset_dims=(), collapsed_slice_dims=(0,),
                                    start_index_map=(0,))
    return lax.gather(x, idx[:, None], dn, slice_sizes=(1,),
                      mode=lax.GatherScatterMode.PROMISE_IN_BOUNDS)


def _sigmoid(x):
    return 1.0 / (1.0 + jnp.exp(-x))


def _hin2vec_body(start_hbm, end_hbm, path_hbm, node_hbm, ptab_hbm, w_hbm,
                  b_hbm, out_hbm, sidx, eidx, pidx, srows, erows, pwtab,
                  wv, bv, ptmp, ptfull, ostage, pw_shared, sem0, sem1):
    cid = lax.axis_index("c")
    sid = lax.axis_index("s")
    wid = sid * _NC + cid
    base = wid * _BPW

    pltpu.sync_copy(w_hbm, wv)
    pltpu.sync_copy(b_hbm, bv)
    pltpu.sync_copy(start_hbm.at[pl.ds(base, _BPW)], sidx)
    pltpu.sync_copy(end_hbm.at[pl.ds(base, _BPW)], eidx)
    pltpu.sync_copy(path_hbm.at[pl.ds(base, _BPW)], pidx.at[pl.ds(0, _BPW)])

    wregs = [wv[pl.ds(16 * j, _L)] for j in range(_D // _L)]
    bvec = bv[...]
    lane = lax.iota(jnp.int32, _L)
    perm = [(lane + sh) & (_L - 1) for sh in (8, 4, 2, 1)]

    sems = [sem0, sem1]

    def make_copies(c, half):
        sl = pl.ds(c * _C, _C)
        hs = pl.ds(half * _C, _C)
        sem = sems[half]
        return (
            pltpu.make_async_copy(node_hbm.at[sidx.at[sl]], srows.at[hs],
                                  sem),
            pltpu.make_async_copy(node_hbm.at[eidx.at[sl]], erows.at[hs],
                                  sem),
        )

    for cp in make_copies(0, 0):
        cp.start()
    for cp in make_copies(1, 1):
        cp.start()

    prow0 = sid * _PROWS
    pltpu.sync_copy(ptab_hbm, ptfull)
    for r in range(_PROWS):
        rr = jnp.minimum(prow0 + r, _PATH_SIZE - 1)
        for j in range(_D // _L):
            v = ptfull[rr, pl.ds(16 * j, _L)]
            ptmp[r, pl.ds(16 * j, _L)] = _sigmoid(v) * wregs[j]
    pltpu.sync_copy(ptmp, pw_shared.at[pl.ds(prow0, _PROWS)])
    plsc.subcore_barrier()
    pltpu.sync_copy(pw_shared, pwtab)

    def chunk_step(c, half):
        for cp in make_copies(c, half):
            cp.wait()

        @pl.when(c + 2 < _NCHUNK)
        def _():
            for cp in make_copies(c + 2, half):
                cp.start()

        rbase = half * _C

        def group_body(g, _):
            def row_body(i, resvec):
                row = rbase + g * _L + i
                pvec = pidx[pl.ds(c * _C + g * _L + i, _L)]
                prow = pvec[0]
                acc = jnp.zeros((_L,), jnp.float32)
                for j in range(_D // _L):
                    s = srows[row, pl.ds(16 * j, _L)]
                    e = erows[row, pl.ds(16 * j, _L)]
                    pw = pwtab[prow, pl.ds(16 * j, _L)]
                    acc = acc + (s * e) * pw
                for pidx_v in perm:
                    acc = acc + _lane_perm(acc, pidx_v)
                return jnp.where(lane == i, acc, resvec)

            resvec = lax.fori_loop(0, _L, row_body,
                                   jnp.zeros((_L,), jnp.float32))
            outv = _sigmoid(resvec + bvec)
            ostage[pl.ds(c * _C + g * _L, _L)] = outv
            return 0

        lax.fori_loop(0, _C // _L, group_body, 0)

    def pair_body(it, _):
        chunk_step(2 * it, 0)
        chunk_step(2 * it + 1, 1)
        return 0

    lax.fori_loop(0, _NCHUNK // 2, pair_body, 0)

    pltpu.sync_copy(ostage, out_hbm.at[pl.ds(base, _BPW)])


@jax.jit
def _hin2vec_sc(start_i, end_i, path_i, node_table, ptab, w_flat, b_vec):
    mesh = plsc.VectorSubcoreMesh(core_axis_name="c", subcore_axis_name="s")
    f = functools.partial(
        pl.kernel,
        mesh=mesh,
        out_type=jax.ShapeDtypeStruct((_B,), jnp.float32),
        scratch_types=[
            pltpu.VMEM((_BPW,), jnp.int32),
            pltpu.VMEM((_BPW,), jnp.int32),
            pltpu.VMEM((_BPW + _L,), jnp.int32),
            pltpu.VMEM((2 * _C, _D), jnp.float32),
            pltpu.VMEM((2 * _C, _D), jnp.float32),
            pltpu.VMEM((_PPAD, _D), jnp.float32),
            pltpu.VMEM((_D,), jnp.float32),
            pltpu.VMEM((_L,), jnp.float32),
            pltpu.VMEM((_PROWS, _D), jnp.float32),
            pltpu.VMEM((_PATH_SIZE, _D), jnp.float32),
            pltpu.VMEM((_BPW,), jnp.float32),
            pltpu.VMEM_SHARED((_PPAD, _D), jnp.float32),
            pltpu.SemaphoreType.DMA,
            pltpu.SemaphoreType.DMA,
        ],
    )(_hin2vec_body)
    return f(start_i, end_i, path_i, node_table, ptab, w_flat, b_vec)


def _as_i32(x):
    return x if x.dtype == jnp.int32 else x.astype(jnp.int32)


def kernel(start_node, end_node, path, node_table, path_table, W, b):
    w_flat = W.reshape(_D)
    b_vec = jnp.broadcast_to(b.reshape(()), (_L,)).astype(jnp.float32)
    out = _hin2vec_sc(_as_i32(start_node), _as_i32(end_node), _as_i32(path),
                      node_table, path_table, w_flat, b_vec)
    return out.reshape(_B, 1)


# speedup vs baseline: 1.1117x; 1.0874x over previous
import functools

import jax
import jax.numpy as jnp
from jax import lax
from jax.experimental import pallas as pl
from jax.experimental.pallas import tpu as pltpu
from jax.experimental.pallas import tpu_sc as plsc

_NODE_SIZE = 100000
_PATH_SIZE = 100
_D = 128
_B = 16384
_L = 16
_NC, _NS = 2, 16
_NW = _NC * _NS
_BPW = _B // _NW
_C = 128
_NCHUNK = _BPW // _C
_PROWS = 8
_PPAD = _PROWS * _NS


def _lane_perm(x, idx):
    dn = lax.GatherDimensionNumbers(offset_dims=(), collapsed_slice_dims=(0,),
                                    start_index_map=(0,))
    return lax.gather(x, idx[:, None], dn, slice_sizes=(1,),
                      mode=lax.GatherScatterMode.PROMISE_IN_BOUNDS)


def _sigmoid(x):
    return 1.0 / (1.0 + jnp.exp(-x))


def _hin2vec_body(start_hbm, end_hbm, path_hbm, node_hbm, ptab_hbm, w_hbm,
                  b_hbm, out_hbm, sidx, eidx, pidx, srows, erows, pwtab,
                  wv, bv, ptmp, ptfull, ostage, pw_shared, sem0, sem1):
    cid = lax.axis_index("c")
    sid = lax.axis_index("s")
    wid = sid * _NC + cid
    base = wid * _BPW

    prologue = [
        pltpu.make_async_copy(w_hbm, wv, sem0),
        pltpu.make_async_copy(b_hbm, bv, sem0),
        pltpu.make_async_copy(ptab_hbm, ptfull, sem0),
        pltpu.make_async_copy(path_hbm.at[pl.ds(base, _BPW)],
                              pidx.at[pl.ds(0, _BPW)], sem0),
    ]
    for c in range(_NCHUNK):
        sl = pl.ds(base + c * _C, _C)
        prologue.append(pltpu.make_async_copy(start_hbm.at[sl], sidx.at[c],
                                              sem0))
        prologue.append(pltpu.make_async_copy(end_hbm.at[sl], eidx.at[c],
                                              sem0))
    for cp in prologue:
        cp.start()
    for cp in prologue:
        cp.wait()

    wregs = [wv[pl.ds(16 * j, _L)] for j in range(_D // _L)]
    bvec = bv[...]
    lane = lax.iota(jnp.int32, _L)
    perm = [(lane + sh) & (_L - 1) for sh in (8, 4, 2, 1)]

    sems = [sem0, sem1]

    def make_copies(c, half):
        hs = pl.ds(half * _C, _C)
        sem = sems[half]
        return (
            pltpu.make_async_copy(node_hbm.at[sidx.at[c]], srows.at[hs],
                                  sem),
            pltpu.make_async_copy(node_hbm.at[eidx.at[c]], erows.at[hs],
                                  sem),
        )

    for cp in make_copies(0, 0):
        cp.start()
    for cp in make_copies(1, 1):
        cp.start()

    prow0 = sid * _PROWS
    for r in range(_PROWS):
        rr = jnp.minimum(prow0 + r, _PATH_SIZE - 1)
        for j in range(_D // _L):
            v = ptfull[rr, pl.ds(16 * j, _L)]
            ptmp[r, pl.ds(16 * j, _L)] = _sigmoid(v) * wregs[j]
    pltpu.sync_copy(ptmp, pw_shared.at[pl.ds(prow0, _PROWS)])
    plsc.subcore_barrier()
    pltpu.sync_copy(pw_shared, pwtab)

    def chunk_step(c, half):
        for cp in make_copies(c, half):
            cp.wait()

        @pl.when(c + 2 < _NCHUNK)
        def _():
            for cp in make_copies(c + 2, half):
                cp.start()

        rbase = half * _C

        def group_body(g, _):
            def row_body(i, resvec):
                row = rbase + g * _L + i
                pvec = pidx[pl.ds(c * _C + g * _L + i, _L)]
                prow = pvec[0]
                acc = jnp.zeros((_L,), jnp.float32)
                for j in range(_D // _L):
                    s = srows[row, pl.ds(16 * j, _L)]
                    e = erows[row, pl.ds(16 * j, _L)]
                    pw = pwtab[prow, pl.ds(16 * j, _L)]
                    acc = acc + (s * e) * pw
                for pidx_v in perm:
                    acc = acc + _lane_perm(acc, pidx_v)
                return jnp.where(lane == i, acc, resvec)

            resvec = lax.fori_loop(0, _L, row_body,
                                   jnp.zeros((_L,), jnp.float32))
            outv = _sigmoid(resvec + bvec)
            ostage[pl.ds(c * _C + g * _L, _L)] = outv
            return 0

        lax.fori_loop(0, _C // _L, group_body, 0)

    def pair_body(it, _):
        chunk_step(2 * it, 0)
        chunk_step(2 * it + 1, 1)
        return 0

    lax.fori_loop(0, _NCHUNK // 2, pair_body, 0)

    pltpu.sync_copy(ostage, out_hbm.at[pl.ds(base, _BPW)])


@jax.jit
def _hin2vec_sc(start_i, end_i, path_i, node_table, ptab, w_flat, b_vec):
    mesh = plsc.VectorSubcoreMesh(core_axis_name="c", subcore_axis_name="s")
    f = functools.partial(
        pl.kernel,
        mesh=mesh,
        out_type=jax.ShapeDtypeStruct((_B,), jnp.float32),
        scratch_types=[
            pltpu.VMEM((_NCHUNK, _C), jnp.int32),
            pltpu.VMEM((_NCHUNK, _C), jnp.int32),
            pltpu.VMEM((_BPW + _L,), jnp.int32),
            pltpu.VMEM((2 * _C, _D), jnp.float32),
            pltpu.VMEM((2 * _C, _D), jnp.float32),
            pltpu.VMEM((_PPAD, _D), jnp.float32),
            pltpu.VMEM((_D,), jnp.float32),
            pltpu.VMEM((_L,), jnp.float32),
            pltpu.VMEM((_PROWS, _D), jnp.float32),
            pltpu.VMEM((_PATH_SIZE, _D), jnp.float32),
            pltpu.VMEM((_BPW,), jnp.float32),
            pltpu.VMEM_SHARED((_PPAD, _D), jnp.float32),
            pltpu.SemaphoreType.DMA,
            pltpu.SemaphoreType.DMA,
        ],
    )(_hin2vec_body)
    return f(start_i, end_i, path_i, node_table, ptab, w_flat, b_vec)


def _as_i32(x):
    return x if x.dtype == jnp.int32 else x.astype(jnp.int32)


def kernel(start_node, end_node, path, node_table, path_table, W, b):
    w_flat = W.reshape(_D)
    b_vec = jnp.broadcast_to(b.reshape(()), (_L,)).astype(jnp.float32)
    out = _hin2vec_sc(_as_i32(start_node), _as_i32(end_node), _as_i32(path),
                      node_table, path_table, w_flat, b_vec)
    return out.reshape(_B, 1)


# speedup vs baseline: 1.1290x; 1.0156x over previous
import functools

import jax
import jax.numpy as jnp
from jax import lax
from jax.experimental import pallas as pl
from jax.experimental.pallas import tpu as pltpu
from jax.experimental.pallas import tpu_sc as plsc

_NODE_SIZE = 100000
_PATH_SIZE = 100
_D = 128
_B = 16384
_L = 16
_NC, _NS = 2, 16
_NW = _NC * _NS
_BPW = _B // _NW
_C = 128
_NCHUNK = _BPW // _C
_PROWS = 8
_PPAD = _PROWS * _NS


def _lane_perm(x, idx):
    dn = lax.GatherDimensionNumbers(offset_dims=(), collapsed_slice_dims=(0,),
                                    start_index_map=(0,))
    return lax.gather(x, idx[:, None], dn, slice_sizes=(1,),
                      mode=lax.GatherScatterMode.PROMISE_IN_BOUNDS)


def _sigmoid(x):
    return 1.0 / (1.0 + jnp.exp(-x))


def _hin2vec_body(start_hbm, end_hbm, path_hbm, node_hbm, ptab_hbm, w_hbm,
                  b_hbm, out_hbm, sidx, eidx, pidx, srows, erows, pwtab,
                  wv, bv, ptmp, ptfull, ostage, pw_shared, sem0, sem1):
    cid = lax.axis_index("c")
    sid = lax.axis_index("s")
    wid = sid * _NC + cid
    base = wid * _BPW

    prologue = [
        pltpu.make_async_copy(w_hbm, wv, sem0),
        pltpu.make_async_copy(b_hbm, bv, sem0),
        pltpu.make_async_copy(ptab_hbm, ptfull, sem0),
        pltpu.make_async_copy(path_hbm.at[pl.ds(base, _BPW)],
                              pidx.at[pl.ds(0, _BPW)], sem0),
    ]
    for c in range(_NCHUNK):
        sl = pl.ds(base + c * _C, _C)
        prologue.append(pltpu.make_async_copy(start_hbm.at[sl], sidx.at[c],
                                              sem0))
        prologue.append(pltpu.make_async_copy(end_hbm.at[sl], eidx.at[c],
                                              sem0))
    for cp in prologue:
        cp.start()
    for cp in prologue:
        cp.wait()

    wregs = [wv[pl.ds(16 * j, _L)] for j in range(_D // _L)]
    bvec = bv[...]
    lane = lax.iota(jnp.int32, _L)
    perm = [(lane + sh) & (_L - 1) for sh in (8, 4, 2, 1)]

    sems = [sem0, sem1]

    def make_copies(c, half):
        hs = pl.ds(half * _C, _C)
        sem = sems[half]
        return (
            pltpu.make_async_copy(node_hbm.at[sidx.at[c]], srows.at[hs],
                                  sem),
            pltpu.make_async_copy(node_hbm.at[eidx.at[c]], erows.at[hs],
                                  sem),
        )

    for cp in make_copies(0, 0):
        cp.start()
    for cp in make_copies(1, 1):
        cp.start()

    prow0 = sid * _PROWS
    for r in range(_PROWS):
        rr = jnp.minimum(prow0 + r, _PATH_SIZE - 1)
        for j in range(_D // _L):
            v = ptfull[rr, pl.ds(16 * j, _L)]
            ptmp[r, pl.ds(16 * j, _L)] = _sigmoid(v) * wregs[j]
    pltpu.sync_copy(ptmp, pw_shared.at[pl.ds(prow0, _PROWS)])
    plsc.subcore_barrier()
    pltpu.sync_copy(pw_shared, pwtab)

    def chunk_step(c, half):
        for cp in make_copies(c, half):
            cp.wait()

        rbase = half * _C

        def group_body(g, _):
            def row_body(i, resvec):
                row = rbase + g * _L + i
                pvec = pidx[pl.ds(c * _C + g * _L + i, _L)]
                prow = pvec[0]
                acc = jnp.zeros((_L,), jnp.float32)
                for j in range(_D // _L):
                    s = srows[row, pl.ds(16 * j, _L)]
                    e = erows[row, pl.ds(16 * j, _L)]
                    pw = pwtab[prow, pl.ds(16 * j, _L)]
                    acc = acc + (s * e) * pw
                for pidx_v in perm:
                    acc = acc + _lane_perm(acc, pidx_v)
                return jnp.where(lane == i, acc, resvec)

            resvec = lax.fori_loop(0, _L, row_body,
                                   jnp.zeros((_L,), jnp.float32))
            outv = _sigmoid(resvec + bvec)
            ostage[pl.ds(c * _C + g * _L, _L)] = outv
            return 0

        lax.fori_loop(0, _C // _L, group_body, 0)

        @pl.when(c + 2 < _NCHUNK)
        def _():
            for cp in make_copies(c + 2, half):
                cp.start()

    def pair_body(it, _):
        chunk_step(2 * it, 0)
        chunk_step(2 * it + 1, 1)
        return 0

    lax.fori_loop(0, _NCHUNK // 2, pair_body, 0)

    pltpu.sync_copy(ostage, out_hbm.at[pl.ds(base, _BPW)])


@jax.jit
def _hin2vec_sc(start_i, end_i, path_i, node_table, ptab, w_flat, b_vec):
    mesh = plsc.VectorSubcoreMesh(core_axis_name="c", subcore_axis_name="s")
    f = functools.partial(
        pl.kernel,
        mesh=mesh,
        out_type=jax.ShapeDtypeStruct((_B,), jnp.float32),
        scratch_types=[
            pltpu.VMEM((_NCHUNK, _C), jnp.int32),
            pltpu.VMEM((_NCHUNK, _C), jnp.int32),
            pltpu.VMEM((_BPW + _L,), jnp.int32),
            pltpu.VMEM((2 * _C, _D), jnp.float32),
            pltpu.VMEM((2 * _C, _D), jnp.float32),
            pltpu.VMEM((_PPAD, _D), jnp.float32),
            pltpu.VMEM((_D,), jnp.float32),
            pltpu.VMEM((_L,), jnp.float32),
            pltpu.VMEM((_PROWS, _D), jnp.float32),
            pltpu.VMEM((_PATH_SIZE, _D), jnp.float32),
            pltpu.VMEM((_BPW,), jnp.float32),
            pltpu.VMEM_SHARED((_PPAD, _D), jnp.float32),
            pltpu.SemaphoreType.DMA,
            pltpu.SemaphoreType.DMA,
        ],
    )(_hin2vec_body)
    return f(start_i, end_i, path_i, node_table, ptab, w_flat, b_vec)


def _as_i32(x):
    return x if x.dtype == jnp.int32 else x.astype(jnp.int32)


def kernel(start_node, end_node, path, node_table, path_table, W, b):
    w_flat = W.reshape(_D)
    b_vec = jnp.broadcast_to(b.reshape(()), (_L,)).astype(jnp.float32)
    out = _hin2vec_sc(_as_i32(start_node), _as_i32(end_node), _as_i32(path),
                      node_table, path_table, w_flat, b_vec)
    return out.reshape(_B, 1)


# speedup vs baseline: 1.1526x; 1.0209x over previous
import functools

import jax
import jax.numpy as jnp
from jax import lax
from jax.experimental import pallas as pl
from jax.experimental.pallas import tpu as pltpu
from jax.experimental.pallas import tpu_sc as plsc

_NODE_SIZE = 100000
_PATH_SIZE = 100
_D = 128
_B = 16384
_L = 16
_NC, _NS = 2, 16
_NW = _NC * _NS
_BPW = _B // _NW
_C = 128
_NCHUNK = _BPW // _C
_PROWS = 8
_PPAD = _PROWS * _NS


def _lane_perm(x, idx):
    dn = lax.GatherDimensionNumbers(offset_dims=(), collapsed_slice_dims=(0,),
                                    start_index_map=(0,))
    return lax.gather(x, idx[:, None], dn, slice_sizes=(1,),
                      mode=lax.GatherScatterMode.PROMISE_IN_BOUNDS)


def _sigmoid(x):
    return 1.0 / (1.0 + jnp.exp(-x))


def _hin2vec_body(start_hbm, end_hbm, path_hbm, node_hbm, ptab_hbm, w_hbm,
                  b_hbm, out_hbm, sidx, eidx, pidx, srows, erows, pwtab,
                  wv, bv, ptmp, ptfull, ostage, pw_shared, sem0, sem1):
    cid = lax.axis_index("c")
    sid = lax.axis_index("s")
    wid = sid * _NC + cid
    base = wid * _BPW

    prologue = [
        pltpu.make_async_copy(w_hbm, wv, sem0),
        pltpu.make_async_copy(b_hbm, bv, sem0),
        pltpu.make_async_copy(ptab_hbm, ptfull, sem0),
        pltpu.make_async_copy(path_hbm.at[pl.ds(base, _BPW)],
                              pidx.at[pl.ds(0, _BPW)], sem0),
    ]
    for c in range(_NCHUNK):
        sl = pl.ds(base + c * _C, _C)
        prologue.append(pltpu.make_async_copy(start_hbm.at[sl], sidx.at[c],
                                              sem0))
        prologue.append(pltpu.make_async_copy(end_hbm.at[sl], eidx.at[c],
                                              sem0))
    for cp in prologue:
        cp.start()
    for cp in prologue:
        cp.wait()

    wregs = [wv[pl.ds(16 * j, _L)] for j in range(_D // _L)]
    bvec = bv[...]
    lane = lax.iota(jnp.int32, _L)
    perm = [(lane + sh) & (_L - 1) for sh in (8, 4, 2, 1)]

    sems = [sem0, sem1]

    def make_copies(c, half):
        hs = pl.ds(half * _C, _C)
        sem = sems[half]
        return (
            pltpu.make_async_copy(node_hbm.at[sidx.at[c]], srows.at[hs],
                                  sem),
            pltpu.make_async_copy(node_hbm.at[eidx.at[c]], erows.at[hs],
                                  sem),
        )

    for cp in make_copies(0, 0):
        cp.start()
    for cp in make_copies(1, 1):
        cp.start()

    prow0 = sid * _PROWS

    def pw_row(r, _):
        rr = jnp.minimum(prow0 + r, _PATH_SIZE - 1)
        for j in range(_D // _L):
            v = ptfull[rr, pl.ds(16 * j, _L)]
            ptmp[r, pl.ds(16 * j, _L)] = _sigmoid(v) * wregs[j]
        return 0

    lax.fori_loop(0, _PROWS, pw_row, 0)
    pltpu.sync_copy(ptmp, pw_shared.at[pl.ds(prow0, _PROWS)])
    plsc.subcore_barrier()
    pltpu.sync_copy(pw_shared, pwtab)

    def chunk_step(c, half):
        for cp in make_copies(c, half):
            cp.wait()

        rbase = half * _C

        def group_body(g, _):
            def row_body(i, resvec):
                row = rbase + g * _L + i
                pvec = pidx[pl.ds(c * _C + g * _L + i, _L)]
                prow = pvec[0]
                acc = jnp.zeros((_L,), jnp.float32)
                for j in range(_D // _L):
                    s = srows[row, pl.ds(16 * j, _L)]
                    e = erows[row, pl.ds(16 * j, _L)]
                    pw = pwtab[prow, pl.ds(16 * j, _L)]
                    acc = acc + (s * e) * pw
                for pidx_v in perm:
                    acc = acc + _lane_perm(acc, pidx_v)
                return jnp.where(lane == i, acc, resvec)

            resvec = lax.fori_loop(0, _L, row_body,
                                   jnp.zeros((_L,), jnp.float32))
            outv = _sigmoid(resvec + bvec)
            ostage[pl.ds(c * _C + g * _L, _L)] = outv
            return 0

        lax.fori_loop(0, _C // _L, group_body, 0)

        @pl.when(c + 2 < _NCHUNK)
        def _():
            for cp in make_copies(c + 2, half):
                cp.start()

    def pair_body(it, _):
        chunk_step(2 * it, 0)
        chunk_step(2 * it + 1, 1)
        return 0

    lax.fori_loop(0, _NCHUNK // 2, pair_body, 0)

    pltpu.sync_copy(ostage, out_hbm.at[pl.ds(base, _BPW)])


@jax.jit
def _hin2vec_sc(start_i, end_i, path_i, node_table, ptab, w_flat, b_vec):
    mesh = plsc.VectorSubcoreMesh(core_axis_name="c", subcore_axis_name="s")
    f = functools.partial(
        pl.kernel,
        mesh=mesh,
        out_type=jax.ShapeDtypeStruct((_B,), jnp.float32),
        scratch_types=[
            pltpu.VMEM((_NCHUNK, _C), jnp.int32),
            pltpu.VMEM((_NCHUNK, _C), jnp.int32),
            pltpu.VMEM((_BPW + _L,), jnp.int32),
            pltpu.VMEM((2 * _C, _D), jnp.float32),
            pltpu.VMEM((2 * _C, _D), jnp.float32),
            pltpu.VMEM((_PPAD, _D), jnp.float32),
            pltpu.VMEM((_D,), jnp.float32),
            pltpu.VMEM((_L,), jnp.float32),
            pltpu.VMEM((_PROWS, _D), jnp.float32),
            pltpu.VMEM((_PATH_SIZE, _D), jnp.float32),
            pltpu.VMEM((_BPW,), jnp.float32),
            pltpu.VMEM_SHARED((_PPAD, _D), jnp.float32),
            pltpu.SemaphoreType.DMA,
            pltpu.SemaphoreType.DMA,
        ],
    )(_hin2vec_body)
    return f(start_i, end_i, path_i, node_table, ptab, w_flat, b_vec)


def _as_i32(x):
    return x if x.dtype == jnp.int32 else x.astype(jnp.int32)


def kernel(start_node, end_node, path, node_table, path_table, W, b):
    w_flat = W.reshape(_D)
    b_vec = jnp.broadcast_to(b.reshape(()), (_L,)).astype(jnp.float32)
    out = _hin2vec_sc(_as_i32(start_node), _as_i32(end_node), _as_i32(path),
                      node_table, path_table, w_flat, b_vec)
    return out.reshape(_B, 1)


# speedup vs baseline: 1.1533x; 1.0006x over previous
import functools

import jax
import jax.numpy as jnp
from jax import lax
from jax.experimental import pallas as pl
from jax.experimental.pallas import tpu as pltpu
from jax.experimental.pallas import tpu_sc as plsc

_NODE_SIZE = 100000
_PATH_SIZE = 100
_D = 128
_B = 16384
_L = 16
_NC, _NS = 2, 16
_NW = _NC * _NS
_BPW = _B // _NW
_C = 128
_NCHUNK = _BPW // _C
_PROWS = 8
_PPAD = _PROWS * _NS


def _lane_perm(x, idx):
    dn = lax.GatherDimensionNumbers(offset_dims=(), collapsed_slice_dims=(0,),
                                    start_index_map=(0,))
    return lax.gather(x, idx[:, None], dn, slice_sizes=(1,),
                      mode=lax.GatherScatterMode.PROMISE_IN_BOUNDS)


def _sigmoid(x):
    return 1.0 / (1.0 + jnp.exp(-x))


def _hin2vec_body(start_hbm, end_hbm, path_hbm, node_hbm, ptab_hbm, w_hbm,
                  b_hbm, out_hbm, sidx, eidx, pidx, srows, erows, pwtab,
                  wv, bv, ptmp, ptfull, ostage, pw_shared, sem0, sem1):
    cid = lax.axis_index("c")
    sid = lax.axis_index("s")
    wid = sid * _NC + cid
    base = wid * _BPW

    prologue = [
        pltpu.make_async_copy(w_hbm, wv, sem0),
        pltpu.make_async_copy(b_hbm, bv, sem0),
        pltpu.make_async_copy(ptab_hbm, ptfull, sem0),
        pltpu.make_async_copy(path_hbm.at[pl.ds(base, _BPW)],
                              pidx.at[pl.ds(0, _BPW)], sem0),
    ]
    for c in range(_NCHUNK):
        sl = pl.ds(base + c * _C, _C)
        prologue.append(pltpu.make_async_copy(start_hbm.at[sl], sidx.at[c],
                                              sem0))
        prologue.append(pltpu.make_async_copy(end_hbm.at[sl], eidx.at[c],
                                              sem0))
    for cp in prologue:
        cp.start()
    for cp in prologue:
        cp.wait()

    wregs = [wv[pl.ds(16 * j, _L)] for j in range(_D // _L)]
    bvec = bv[...]
    lane = lax.iota(jnp.int32, _L)
    perm = [(lane + sh) & (_L - 1) for sh in (8, 4, 2, 1)]

    sems = [sem0, sem1]

    def make_copies(c, half):
        hs = pl.ds(half * _C, _C)
        sem = sems[half]
        return (
            pltpu.make_async_copy(node_hbm.at[sidx.at[c]], srows.at[hs],
                                  sem),
            pltpu.make_async_copy(node_hbm.at[eidx.at[c]], erows.at[hs],
                                  sem),
        )

    for cp in make_copies(0, 0):
        cp.start()
    for cp in make_copies(1, 1):
        cp.start()

    prow0 = sid * _PROWS

    def pw_row(r, _):
        rr = jnp.minimum(prow0 + r, _PATH_SIZE - 1)
        for j in range(_D // _L):
            v = ptfull[rr, pl.ds(16 * j, _L)]
            ptmp[r, pl.ds(16 * j, _L)] = _sigmoid(v) * wregs[j]
        return 0

    lax.fori_loop(0, _PROWS, pw_row, 0)
    pltpu.sync_copy(ptmp, pw_shared.at[pl.ds(prow0, _PROWS)])
    plsc.subcore_barrier()
    pltpu.sync_copy(pw_shared, pwtab)

    def chunk_step(c, half):
        for cp in make_copies(c, half):
            cp.wait()

        rbase = half * _C

        def group_body(g, _):
            def row_body(i, resvec):
                row = rbase + g * _L + i
                pvec = pidx[pl.ds(c * _C + g * _L + i, _L)]
                prow = pvec[0]
                acc = jnp.zeros((_L,), jnp.float32)
                for j in range(_D // _L):
                    s = srows[row, pl.ds(16 * j, _L)]
                    e = erows[row, pl.ds(16 * j, _L)]
                    pw = pwtab[prow, pl.ds(16 * j, _L)]
                    acc = acc + (s * e) * pw
                for pidx_v in perm:
                    acc = acc + _lane_perm(acc, pidx_v)
                return jnp.where(lane == i, acc, resvec)

            resvec = lax.fori_loop(0, _L, row_body,
                                   jnp.zeros((_L,), jnp.float32), unroll=2)
            outv = _sigmoid(resvec + bvec)
            ostage[pl.ds(c * _C + g * _L, _L)] = outv
            return 0

        lax.fori_loop(0, _C // _L, group_body, 0)

        @pl.when(c + 2 < _NCHUNK)
        def _():
            for cp in make_copies(c + 2, half):
                cp.start()

    def pair_body(it, _):
        chunk_step(2 * it, 0)
        chunk_step(2 * it + 1, 1)
        return 0

    lax.fori_loop(0, _NCHUNK // 2, pair_body, 0)

    pltpu.sync_copy(ostage, out_hbm.at[pl.ds(base, _BPW)])


@jax.jit
def _hin2vec_sc(start_i, end_i, path_i, node_table, ptab, w_flat, b_vec):
    mesh = plsc.VectorSubcoreMesh(core_axis_name="c", subcore_axis_name="s")
    f = functools.partial(
        pl.kernel,
        mesh=mesh,
        out_type=jax.ShapeDtypeStruct((_B,), jnp.float32),
        scratch_types=[
            pltpu.VMEM((_NCHUNK, _C), jnp.int32),
            pltpu.VMEM((_NCHUNK, _C), jnp.int32),
            pltpu.VMEM((_BPW + _L,), jnp.int32),
            pltpu.VMEM((2 * _C, _D), jnp.float32),
            pltpu.VMEM((2 * _C, _D), jnp.float32),
            pltpu.VMEM((_PPAD, _D), jnp.float32),
            pltpu.VMEM((_D,), jnp.float32),
            pltpu.VMEM((_L,), jnp.float32),
            pltpu.VMEM((_PROWS, _D), jnp.float32),
            pltpu.VMEM((_PATH_SIZE, _D), jnp.float32),
            pltpu.VMEM((_BPW,), jnp.float32),
            pltpu.VMEM_SHARED((_PPAD, _D), jnp.float32),
            pltpu.SemaphoreType.DMA,
            pltpu.SemaphoreType.DMA,
        ],
    )(_hin2vec_body)
    return f(start_i, end_i, path_i, node_table, ptab, w_flat, b_vec)


def _as_i32(x):
    return x if x.dtype == jnp.int32 else x.astype(jnp.int32)


def kernel(start_node, end_node, path, node_table, path_table, W, b):
    w_flat = W.reshape(_D)
    b_vec = jnp.broadcast_to(b.reshape(()), (_L,)).astype(jnp.float32)
    out = _hin2vec_sc(_as_i32(start_node), _as_i32(end_node), _as_i32(path),
                      node_table, path_table, w_flat, b_vec)
    return out.reshape(_B, 1)


# speedup vs baseline: 1.1832x; 1.0259x over previous
import functools

import jax
import jax.numpy as jnp
from jax import lax
from jax.experimental import pallas as pl
from jax.experimental.pallas import tpu as pltpu
from jax.experimental.pallas import tpu_sc as plsc

_NODE_SIZE = 100000
_PATH_SIZE = 100
_D = 128
_B = 16384
_L = 16
_NC, _NS = 2, 16
_NW = _NC * _NS
_BPW = _B // _NW
_C = 128
_NCHUNK = _BPW // _C
_PROWS = 8
_PPAD = _PROWS * _NS


def _lane_perm(x, idx):
    dn = lax.GatherDimensionNumbers(offset_dims=(), collapsed_slice_dims=(0,),
                                    start_index_map=(0,))
    return lax.gather(x, idx[:, None], dn, slice_sizes=(1,),
                      mode=lax.GatherScatterMode.PROMISE_IN_BOUNDS)


def _sigmoid(x):
    return 1.0 / (1.0 + jnp.exp(-x))


def _hin2vec_body(start_hbm, end_hbm, path_hbm, node_hbm, ptab_hbm, w_hbm,
                  b_hbm, out_hbm, sidx, eidx, pidx, srows, erows, pwtab,
                  wv, bv, ptmp, ptfull, ostage, pw_shared, sem0, sem1):
    cid = lax.axis_index("c")
    sid = lax.axis_index("s")
    wid = sid * _NC + cid
    base = wid * _BPW

    prologue = [
        pltpu.make_async_copy(w_hbm, wv, sem0),
        pltpu.make_async_copy(b_hbm, bv, sem0),
        pltpu.make_async_copy(ptab_hbm, ptfull, sem0),
        pltpu.make_async_copy(path_hbm.at[pl.ds(base, _BPW)],
                              pidx.at[pl.ds(0, _BPW)], sem0),
    ]
    for c in range(_NCHUNK):
        sl = pl.ds(base + c * _C, _C)
        prologue.append(pltpu.make_async_copy(start_hbm.at[sl], sidx.at[c],
                                              sem0))
        prologue.append(pltpu.make_async_copy(end_hbm.at[sl], eidx.at[c],
                                              sem0))
    for cp in prologue:
        cp.start()
    for cp in prologue:
        cp.wait()

    wregs = [wv[pl.ds(16 * j, _L)] for j in range(_D // _L)]
    bvec = bv[...]
    lane = lax.iota(jnp.int32, _L)
    perm = [(lane + sh) & (_L - 1) for sh in (8, 4, 2, 1)]

    sems = [sem0, sem1]

    def make_copies(c, half):
        hs = pl.ds(half * _C, _C)
        sem = sems[half]
        return (
            pltpu.make_async_copy(node_hbm.at[sidx.at[c]], srows.at[hs],
                                  sem),
            pltpu.make_async_copy(node_hbm.at[eidx.at[c]], erows.at[hs],
                                  sem),
        )

    for cp in make_copies(0, 0):
        cp.start()
    for cp in make_copies(1, 1):
        cp.start()

    prow0 = sid * _PROWS

    half_bit = jnp.full((_L,), 0x8000, jnp.uint32)
    hi_mask = jnp.full((_L,), 0xFFFF0000, jnp.uint32)
    lo_mask = jnp.full((_L,), 0x0000FFFF, jnp.uint32)

    def pw_row(r, _):
        rr = jnp.minimum(prow0 + r, _PATH_SIZE - 1)
        for jj in range(_D // (2 * _L)):
            va = ptfull[rr, pl.ds(32 * jj, _L)]
            vb = ptfull[rr, pl.ds(32 * jj + _L, _L)]
            pa = _sigmoid(va) * wregs[2 * jj]
            pb = _sigmoid(vb) * wregs[2 * jj + 1]
            ua = lax.bitcast_convert_type(pa, jnp.uint32)
            ub = lax.bitcast_convert_type(pb, jnp.uint32)
            packed = ((ua + half_bit) & hi_mask) | (
                ((ub + half_bit) >> 16) & lo_mask)
            ptmp[r, pl.ds(16 * jj, _L)] = packed
        return 0

    lax.fori_loop(0, _PROWS, pw_row, 0)
    pltpu.sync_copy(ptmp, pw_shared.at[pl.ds(prow0, _PROWS)])
    plsc.subcore_barrier()
    pltpu.sync_copy(pw_shared, pwtab)

    def chunk_step(c, half):
        for cp in make_copies(c, half):
            cp.wait()

        rbase = half * _C

        def group_body(g, _):
            def row_body(i, resvec):
                row = rbase + g * _L + i
                pvec = pidx[pl.ds(c * _C + g * _L + i, _L)]
                prow = pvec[0]
                acc = jnp.zeros((_L,), jnp.float32)
                for jj in range(_D // (2 * _L)):
                    pwp = pwtab[prow, pl.ds(16 * jj, _L)]
                    pa = lax.bitcast_convert_type(pwp & hi_mask, jnp.float32)
                    pb = lax.bitcast_convert_type(pwp << 16, jnp.float32)
                    sa = srows[row, pl.ds(32 * jj, _L)]
                    ea = erows[row, pl.ds(32 * jj, _L)]
                    sb = srows[row, pl.ds(32 * jj + _L, _L)]
                    eb = erows[row, pl.ds(32 * jj + _L, _L)]
                    acc = acc + (sa * ea) * pa + (sb * eb) * pb
                for pidx_v in perm:
                    acc = acc + _lane_perm(acc, pidx_v)
                return jnp.where(lane == i, acc, resvec)

            resvec = lax.fori_loop(0, _L, row_body,
                                   jnp.zeros((_L,), jnp.float32))
            outv = _sigmoid(resvec + bvec)
            ostage[pl.ds(c * _C + g * _L, _L)] = outv
            return 0

        lax.fori_loop(0, _C // _L, group_body, 0)

        @pl.when(c + 2 < _NCHUNK)
        def _():
            for cp in make_copies(c + 2, half):
                cp.start()

    def pair_body(it, _):
        chunk_step(2 * it, 0)
        chunk_step(2 * it + 1, 1)
        return 0

    lax.fori_loop(0, _NCHUNK // 2, pair_body, 0)

    pltpu.sync_copy(ostage, out_hbm.at[pl.ds(base, _BPW)])


@jax.jit
def _hin2vec_sc(start_i, end_i, path_i, node_table, ptab, w_flat, b_vec):
    mesh = plsc.VectorSubcoreMesh(core_axis_name="c", subcore_axis_name="s")
    f = functools.partial(
        pl.kernel,
        mesh=mesh,
        out_type=jax.ShapeDtypeStruct((_B,), jnp.float32),
        scratch_types=[
            pltpu.VMEM((_NCHUNK, _C), jnp.int32),
            pltpu.VMEM((_NCHUNK, _C), jnp.int32),
            pltpu.VMEM((_BPW + _L,), jnp.int32),
            pltpu.VMEM((2 * _C, _D), jnp.float32),
            pltpu.VMEM((2 * _C, _D), jnp.float32),
            pltpu.VMEM((_PPAD, _D // 2), jnp.uint32),
            pltpu.VMEM((_D,), jnp.float32),
            pltpu.VMEM((_L,), jnp.float32),
            pltpu.VMEM((_PROWS, _D // 2), jnp.uint32),
            pltpu.VMEM((_PATH_SIZE, _D), jnp.float32),
            pltpu.VMEM((_BPW,), jnp.float32),
            pltpu.VMEM_SHARED((_PPAD, _D // 2), jnp.uint32),
            pltpu.SemaphoreType.DMA,
            pltpu.SemaphoreType.DMA,
        ],
    )(_hin2vec_body)
    return f(start_i, end_i, path_i, node_table, ptab, w_flat, b_vec)


def _as_i32(x):
    return x if x.dtype == jnp.int32 else x.astype(jnp.int32)


def kernel(start_node, end_node, path, node_table, path_table, W, b):
    w_flat = W.reshape(_D)
    b_vec = jnp.broadcast_to(b.reshape(()), (_L,)).astype(jnp.float32)
    out = _hin2vec_sc(_as_i32(start_node), _as_i32(end_node), _as_i32(path),
                      node_table, path_table, w_flat, b_vec)
    return out.reshape(_B, 1)
